# baseline (speedup 1.0000x reference)
import functools

import jax
import jax.numpy as jnp
from jax import lax
from jax.experimental import pallas as pl
from jax.experimental.pallas import tpu as pltpu
from jax.experimental.pallas import tpu_sc as plsc

N_NODES = 10000
N_EDGES = 320000
E_PAD = 327680
PAD = E_PAD - N_EDGES
TBL_ROWS = 10000
NC = 2
NS = 16
LANES = 16
TILE_ROWS = 624
TAIL_ROWS = 16
CH_ROWS = 8


def _make_seg_sum(nj, src_stride, dst_stride):
    mesh = plsc.VectorSubcoreMesh(
        core_axis_name="c", subcore_axis_name="s",
        num_cores=NC, num_subcores=NS)
    nsub = nj * CH_ROWS

    @functools.partial(
        pl.kernel,
        out_type=jax.ShapeDtypeStruct((NC * N_NODES, 128), jnp.float32),
        mesh=mesh,
        scratch_types=[
            pltpu.VMEM((CH_ROWS, 128), jnp.int32),
            pltpu.VMEM((CH_ROWS, 128), jnp.int32),
            pltpu.VMEM((2, 128, 128), jnp.float32),
            pltpu.VMEM_SHARED((TBL_ROWS, 128), jnp.float32),
            pltpu.SemaphoreType.DMA((2,)),
            pltpu.SemaphoreType.DMA((2,)),
        ],
    )
    def seg_sum(src_tbl, src_idx, dst_idx, out, srcv, dstv, rows, table,
                gsem, ssem):
        c = lax.axis_index("c")
        s = lax.axis_index("s")

        zero16 = jnp.zeros((LANES,), jnp.float32)

        def zrow(r, carry):
            for cc in range(8):
                rows[0, r, pl.ds(cc * LANES, LANES)] = zero16
            return carry

        lax.fori_loop(0, 128, zrow, 0)
        base = s * TILE_ROWS
        for z in range(4):
            pltpu.sync_copy(rows.at[0],
                            table.at[pl.ds(base + z * 128, 128)])
        pltpu.sync_copy(rows.at[0, pl.ds(0, 112)],
                        table.at[pl.ds(base + 512, 112)])

        @pl.when(s == 0)
        def _():
            pltpu.sync_copy(rows.at[0, pl.ds(0, TAIL_ROWS)],
                            table.at[pl.ds(NS * TILE_ROWS, TAIL_ROWS)])

        plsc.subcore_barrier()

        def chunk_body(k, carry):
            srow = c * src_stride + (s * nj + k) * CH_ROWS
            drow = c * dst_stride + (s * nj + k) * CH_ROWS
            pltpu.sync_copy(src_idx.at[pl.ds(srow, CH_ROWS)], srcv)
            pltpu.sync_copy(dst_idx.at[pl.ds(drow, CH_ROWS)], dstv)
            gd = [None] * CH_ROWS
            sd = [None] * CH_ROWS
            gd[0] = pltpu.async_copy(
                src_tbl.at[srcv.at[0]], rows.at[0], gsem.at[0])
            for q in range(CH_ROWS):
                b = q % 2
                gd[q].wait()
                if q >= 1:
                    sd[q - 1].wait()
                if q < CH_ROWS - 1:
                    gd[q + 1] = pltpu.async_copy(
                        src_tbl.at[srcv.at[q + 1]], rows.at[1 - b],
                        gsem.at[1 - b])
                sd[q] = pltpu.async_copy(
                    rows.at[b], table.at[dstv.at[q]], ssem.at[b],
                    add=True)
            sd[CH_ROWS - 1].wait()
            return carry

        lax.fori_loop(0, nj, chunk_body, 0)
        plsc.subcore_barrier()

        pltpu.sync_copy(
            table.at[pl.ds(s * TILE_ROWS, TILE_ROWS)],
            out.at[pl.ds(c * N_NODES + s * TILE_ROWS, TILE_ROWS)])

        @pl.when(s == 0)
        def _():
            pltpu.sync_copy(
                table.at[pl.ds(NS * TILE_ROWS, TAIL_ROWS)],
                out.at[pl.ds(c * N_NODES + NS * TILE_ROWS, TAIL_ROWS)])

    return seg_sum


_seg_sum_l0 = _make_seg_sum(nj=10, src_stride=1280, dst_stride=1280)
_seg_sum_l1 = _make_seg_sum(nj=20, src_stride=2560, dst_stride=0)

_ROW_BLK = 1000


def _mlp0_body(s_ref, x_ref, a_ref, w1_ref, b1_ref, w2_ref, b2_ref, o_ref):
    rst = x_ref[...] * s_ref[0, 0] + a_ref[0] + a_ref[1]
    z = jnp.maximum(
        jnp.dot(rst, w1_ref[...], preferred_element_type=jnp.float32)
        + b1_ref[...], 0.0)
    o_ref[...] = (
        jnp.dot(z, w2_ref[...], preferred_element_type=jnp.float32)
        + b2_ref[...])


def _mlp0(scale, x, agg, W1, b1, W2, b2):
    grid = (N_NODES // _ROW_BLK,)
    return pl.pallas_call(
        _mlp0_body,
        grid=grid,
        in_specs=[
            pl.BlockSpec(memory_space=pltpu.SMEM),
            pl.BlockSpec((_ROW_BLK, 128), lambda i: (i, 0)),
            pl.BlockSpec((2, _ROW_BLK, 128), lambda i: (0, i, 0)),
            pl.BlockSpec((128, 256), lambda i: (0, 0)),
            pl.BlockSpec((1, 256), lambda i: (0, 0)),
            pl.BlockSpec((256, 256), lambda i: (0, 0)),
            pl.BlockSpec((1, 256), lambda i: (0, 0)),
        ],
        out_specs=pl.BlockSpec((_ROW_BLK, 256), lambda i: (i, 0)),
        out_shape=jax.ShapeDtypeStruct((N_NODES, 256), jnp.float32),
    )(scale, x, agg, W1, b1, W2, b2)


def _mlp1_body(s_ref, h_ref, a_ref, w1_ref, b1_ref, w2_ref, b2_ref, o_ref,
               acc_ref):
    i = pl.program_id(0)
    sc = s_ref[0, 0]
    rst = jnp.concatenate(
        [h_ref[:, :128] * sc + a_ref[0], h_ref[:, 128:] * sc + a_ref[1]],
        axis=1)
    z = jnp.maximum(
        jnp.dot(rst, w1_ref[...], preferred_element_type=jnp.float32)
        + b1_ref[...], 0.0)
    part = jnp.sum(z, axis=0, keepdims=True)

    @pl.when(i == 0)
    def _():
        acc_ref[...] = part

    @pl.when(i > 0)
    def _():
        acc_ref[...] += part

    @pl.when(i == pl.num_programs(0) - 1)
    def _():
        o_ref[...] = (
            jnp.dot(acc_ref[...], w2_ref[...],
                    preferred_element_type=jnp.float32)
            + b2_ref[...] * float(N_NODES))


def _mlp1(scale, h, agg, W1, b1, W2, b2):
    grid = (N_NODES // _ROW_BLK,)
    return pl.pallas_call(
        _mlp1_body,
        grid=grid,
        in_specs=[
            pl.BlockSpec(memory_space=pltpu.SMEM),
            pl.BlockSpec((_ROW_BLK, 256), lambda i: (i, 0)),
            pl.BlockSpec((2, _ROW_BLK, 128), lambda i: (0, i, 0)),
            pl.BlockSpec((256, 256), lambda i: (0, 0)),
            pl.BlockSpec((1, 256), lambda i: (0, 0)),
            pl.BlockSpec((256, 256), lambda i: (0, 0)),
            pl.BlockSpec((1, 256), lambda i: (0, 0)),
        ],
        out_specs=pl.BlockSpec((1, 256), lambda i: (0, 0)),
        out_shape=jax.ShapeDtypeStruct((1, 256), jnp.float32),
        scratch_shapes=[pltpu.VMEM((1, 256), jnp.float32)],
    )(scale, h, agg, W1, b1, W2, b2)


def kernel(x, edge_index, eps0, W1_0, b1_0, W2_0, b2_0,
           eps1, W1_1, b1_1, W2_1, b2_1):
    ei = edge_index.astype(jnp.int32)
    ar = jnp.arange(PAD, dtype=jnp.int32)
    src = jnp.concatenate([ei[0], N_NODES + (ar % 4)])
    dst = jnp.concatenate([ei[1], ar % N_NODES])
    src2d = src.reshape(-1, 128)
    dst2d = dst.reshape(-1, 128)

    xz = jnp.concatenate([x, jnp.zeros((8, 128), x.dtype)], axis=0)
    agg0 = _seg_sum_l0(xz, src2d, dst2d).reshape(NC, N_NODES, 128)
    scale0 = (1.0 + eps0).astype(jnp.float32).reshape(1, 1)
    h = _mlp0(scale0, x, agg0, W1_0, b1_0.reshape(1, 256),
              W2_0, b2_0.reshape(1, 256))

    hv = jnp.concatenate(
        [h, jnp.zeros((4, 256), h.dtype)], axis=0).reshape(-1, 128)
    src2 = jnp.concatenate([src * 2, src * 2 + 1]).reshape(-1, 128)
    agg1 = _seg_sum_l1(hv, src2, dst2d).reshape(NC, N_NODES, 128)
    scale1 = (1.0 + eps1).astype(jnp.float32).reshape(1, 1)
    logits = _mlp1(scale1, h, agg1, W1_1, b1_1.reshape(1, 256),
                   W2_1, b2_1.reshape(1, 256))
    return logits


# speedup vs baseline: 1.1780x; 1.1780x over previous
import functools

import jax
import jax.numpy as jnp
from jax import lax
from jax.experimental import pallas as pl
from jax.experimental.pallas import tpu as pltpu
from jax.experimental.pallas import tpu_sc as plsc

N_NODES = 10000
N_EDGES = 320000
E_PAD = 327680
PAD = E_PAD - N_EDGES
TBL_ROWS = 10000
NC = 2
NS = 16
LANES = 16
TILE_ROWS = 624
TAIL_ROWS = 16
CH_ROWS = 16


def _make_seg_sum(nj, src_stride, dst_stride):
    mesh = plsc.VectorSubcoreMesh(
        core_axis_name="c", subcore_axis_name="s",
        num_cores=NC, num_subcores=NS)
    nsub = nj * CH_ROWS

    @functools.partial(
        pl.kernel,
        out_type=jax.ShapeDtypeStruct((NC * N_NODES, 128), jnp.float32),
        mesh=mesh,
        scratch_types=[
            pltpu.VMEM((CH_ROWS, 128), jnp.int32),
            pltpu.VMEM((CH_ROWS, 128), jnp.int32),
            pltpu.VMEM((2, 128, 128), jnp.float32),
            pltpu.VMEM_SHARED((TBL_ROWS, 128), jnp.float32),
            pltpu.SemaphoreType.DMA((2,)),
            pltpu.SemaphoreType.DMA((2,)),
        ],
    )
    def seg_sum(src_tbl, src_idx, dst_idx, out, srcv, dstv, rows, table,
                gsem, ssem):
        c = lax.axis_index("c")
        s = lax.axis_index("s")

        zero16 = jnp.zeros((LANES,), jnp.float32)

        def zrow(r, carry):
            for cc in range(8):
                rows[0, r, pl.ds(cc * LANES, LANES)] = zero16
            return carry

        lax.fori_loop(0, 128, zrow, 0)
        base = s * TILE_ROWS
        for z in range(4):
            pltpu.sync_copy(rows.at[0],
                            table.at[pl.ds(base + z * 128, 128)])
        pltpu.sync_copy(rows.at[0, pl.ds(0, 112)],
                        table.at[pl.ds(base + 512, 112)])

        @pl.when(s == 0)
        def _():
            pltpu.sync_copy(rows.at[0, pl.ds(0, TAIL_ROWS)],
                            table.at[pl.ds(NS * TILE_ROWS, TAIL_ROWS)])

        plsc.subcore_barrier()

        def chunk_body(k, carry):
            ch = s + k * NS
            srow = c * src_stride + ch * CH_ROWS
            drow = c * dst_stride + ch * CH_ROWS
            pltpu.sync_copy(src_idx.at[pl.ds(srow, CH_ROWS)], srcv)
            pltpu.sync_copy(dst_idx.at[pl.ds(drow, CH_ROWS)], dstv)
            gd = [None] * CH_ROWS
            sd = [None] * CH_ROWS
            gd[0] = pltpu.async_copy(
                src_tbl.at[srcv.at[0]], rows.at[0], gsem.at[0])
            for q in range(CH_ROWS):
                b = q % 2
                gd[q].wait()
                if q >= 1:
                    sd[q - 1].wait()
                if q < CH_ROWS - 1:
                    gd[q + 1] = pltpu.async_copy(
                        src_tbl.at[srcv.at[q + 1]], rows.at[1 - b],
                        gsem.at[1 - b])
                sd[q] = pltpu.async_copy(
                    rows.at[b], table.at[dstv.at[q]], ssem.at[b],
                    add=True)
            sd[CH_ROWS - 1].wait()
            return carry

        lax.fori_loop(0, nj, chunk_body, 0)
        plsc.subcore_barrier()

        pltpu.sync_copy(
            table.at[pl.ds(s * TILE_ROWS, TILE_ROWS)],
            out.at[pl.ds(c * N_NODES + s * TILE_ROWS, TILE_ROWS)])

        @pl.when(s == 0)
        def _():
            pltpu.sync_copy(
                table.at[pl.ds(NS * TILE_ROWS, TAIL_ROWS)],
                out.at[pl.ds(c * N_NODES + NS * TILE_ROWS, TAIL_ROWS)])

    return seg_sum


_seg_sum_l0 = _make_seg_sum(nj=5, src_stride=1280, dst_stride=1280)
_seg_sum_l1 = _make_seg_sum(nj=10, src_stride=2560, dst_stride=0)

_ROW_BLK = 1000


def _mlp0_body(s_ref, x_ref, a_ref, w1_ref, b1_ref, w2_ref, b2_ref, o_ref):
    rst = x_ref[...] * s_ref[0, 0] + a_ref[0] + a_ref[1]
    z = jnp.maximum(
        jnp.dot(rst, w1_ref[...], preferred_element_type=jnp.float32)
        + b1_ref[...], 0.0)
    o_ref[...] = (
        jnp.dot(z, w2_ref[...], preferred_element_type=jnp.float32)
        + b2_ref[...])


def _mlp0(scale, x, agg, W1, b1, W2, b2):
    grid = (N_NODES // _ROW_BLK,)
    return pl.pallas_call(
        _mlp0_body,
        grid=grid,
        in_specs=[
            pl.BlockSpec(memory_space=pltpu.SMEM),
            pl.BlockSpec((_ROW_BLK, 128), lambda i: (i, 0)),
            pl.BlockSpec((2, _ROW_BLK, 128), lambda i: (0, i, 0)),
            pl.BlockSpec((128, 256), lambda i: (0, 0)),
            pl.BlockSpec((1, 256), lambda i: (0, 0)),
            pl.BlockSpec((256, 256), lambda i: (0, 0)),
            pl.BlockSpec((1, 256), lambda i: (0, 0)),
        ],
        out_specs=pl.BlockSpec((_ROW_BLK, 256), lambda i: (i, 0)),
        out_shape=jax.ShapeDtypeStruct((N_NODES, 256), jnp.float32),
    )(scale, x, agg, W1, b1, W2, b2)


def _mlp1_body(s_ref, h_ref, a_ref, w1_ref, b1_ref, w2_ref, b2_ref, o_ref,
               acc_ref):
    i = pl.program_id(0)
    sc = s_ref[0, 0]
    rst = jnp.concatenate(
        [h_ref[:, :128] * sc + a_ref[0], h_ref[:, 128:] * sc + a_ref[1]],
        axis=1)
    z = jnp.maximum(
        jnp.dot(rst, w1_ref[...], preferred_element_type=jnp.float32)
        + b1_ref[...], 0.0)
    part = jnp.sum(z, axis=0, keepdims=True)

    @pl.when(i == 0)
    def _():
        acc_ref[...] = part

    @pl.when(i > 0)
    def _():
        acc_ref[...] += part

    @pl.when(i == pl.num_programs(0) - 1)
    def _():
        o_ref[...] = (
            jnp.dot(acc_ref[...], w2_ref[...],
                    preferred_element_type=jnp.float32)
            + b2_ref[...] * float(N_NODES))


def _mlp1(scale, h, agg, W1, b1, W2, b2):
    grid = (N_NODES // _ROW_BLK,)
    return pl.pallas_call(
        _mlp1_body,
        grid=grid,
        in_specs=[
            pl.BlockSpec(memory_space=pltpu.SMEM),
            pl.BlockSpec((_ROW_BLK, 256), lambda i: (i, 0)),
            pl.BlockSpec((2, _ROW_BLK, 128), lambda i: (0, i, 0)),
            pl.BlockSpec((256, 256), lambda i: (0, 0)),
            pl.BlockSpec((1, 256), lambda i: (0, 0)),
            pl.BlockSpec((256, 256), lambda i: (0, 0)),
            pl.BlockSpec((1, 256), lambda i: (0, 0)),
        ],
        out_specs=pl.BlockSpec((1, 256), lambda i: (0, 0)),
        out_shape=jax.ShapeDtypeStruct((1, 256), jnp.float32),
        scratch_shapes=[pltpu.VMEM((1, 256), jnp.float32)],
    )(scale, h, agg, W1, b1, W2, b2)


def kernel(x, edge_index, eps0, W1_0, b1_0, W2_0, b2_0,
           eps1, W1_1, b1_1, W2_1, b2_1):
    ei = edge_index.astype(jnp.int32)
    ar = jnp.arange(PAD, dtype=jnp.int32)
    src = jnp.concatenate([ei[0], N_NODES + (ar % 4)])
    dst = jnp.concatenate([ei[1], ar % N_NODES])
    src2d = src.reshape(-1, 128)
    dst2d = dst.reshape(-1, 128)

    xz = jnp.concatenate([x, jnp.zeros((8, 128), x.dtype)], axis=0)
    agg0 = _seg_sum_l0(xz, src2d, dst2d).reshape(NC, N_NODES, 128)
    scale0 = (1.0 + eps0).astype(jnp.float32).reshape(1, 1)
    h = _mlp0(scale0, x, agg0, W1_0, b1_0.reshape(1, 256),
              W2_0, b2_0.reshape(1, 256))

    hv = jnp.concatenate(
        [h, jnp.zeros((4, 256), h.dtype)], axis=0).reshape(-1, 128)
    src2 = jnp.concatenate([src * 2, src * 2 + 1]).reshape(-1, 128)
    agg1 = _seg_sum_l1(hv, src2, dst2d).reshape(NC, N_NODES, 128)
    scale1 = (1.0 + eps1).astype(jnp.float32).reshape(1, 1)
    logits = _mlp1(scale1, h, agg1, W1_1, b1_1.reshape(1, 256),
                   W2_1, b2_1.reshape(1, 256))
    return logits


# speedup vs baseline: 1.5725x; 1.3348x over previous
import functools

import jax
import jax.numpy as jnp
from jax import lax
from jax.experimental import pallas as pl
from jax.experimental.pallas import tpu as pltpu
from jax.experimental.pallas import tpu_sc as plsc

N_NODES = 10000
N_EDGES = 320000
E_PAD = 327680
PAD = E_PAD - N_EDGES
TBL_ROWS = 10128
NC = 2
NS = 16
LANES = 16
TILE_ROWS = 624
TAIL_ROWS = 16


def _make_seg_sum(nj, ch_rows, src_stride, dst_stride):
    mesh = plsc.VectorSubcoreMesh(
        core_axis_name="c", subcore_axis_name="s",
        num_cores=NC, num_subcores=NS)

    @functools.partial(
        pl.kernel,
        out_type=jax.ShapeDtypeStruct((NC * N_NODES, 128), jnp.float32),
        mesh=mesh,
        scratch_types=[
            pltpu.VMEM((2, ch_rows, 128), jnp.int32),
            pltpu.VMEM((2, ch_rows, 128), jnp.int32),
            pltpu.VMEM((2, 128, 128), jnp.float32),
            pltpu.VMEM_SHARED((TBL_ROWS, 128), jnp.float32),
            pltpu.SemaphoreType.DMA((2,)),
            pltpu.SemaphoreType.DMA((2,)),
        ],
    )
    def seg_sum(src_tbl, src_idx, dst_idx, out, srcv, dstv, rows, table,
                gsem, ssem):
        c = lax.axis_index("c")
        s = lax.axis_index("s")

        zero16 = jnp.zeros((LANES,), jnp.float32)

        def zrow(r, carry):
            for cc in range(8):
                rows[0, r, pl.ds(cc * LANES, LANES)] = zero16
            return carry

        lax.fori_loop(0, 128, zrow, 0)
        base = s * TILE_ROWS
        for z in range(4):
            pltpu.sync_copy(rows.at[0],
                            table.at[pl.ds(base + z * 128, 128)])
        pltpu.sync_copy(rows.at[0, pl.ds(0, 112)],
                        table.at[pl.ds(base + 512, 112)])

        @pl.when(s == 0)
        def _():
            pltpu.sync_copy(rows.at[0, pl.ds(0, TAIL_ROWS)],
                            table.at[pl.ds(NS * TILE_ROWS, TAIL_ROWS)])

        plsc.subcore_barrier()

        def load_idx(k, sl):
            ch = s + k * NS
            srow = c * src_stride + ch * ch_rows
            drow = c * dst_stride + ch * ch_rows
            pltpu.sync_copy(src_idx.at[pl.ds(srow, ch_rows)], srcv.at[sl])
            pltpu.sync_copy(dst_idx.at[pl.ds(drow, ch_rows)], dstv.at[sl])

        load_idx(0, 0)

        def chunk_body(k, carry):
            sl = lax.rem(k, 2)
            gd = [None] * ch_rows
            sd = [None] * ch_rows
            gd[0] = pltpu.async_copy(
                src_tbl.at[srcv.at[sl, 0]], rows.at[0], gsem.at[0])
            for q in range(ch_rows):
                b = q % 2
                gd[q].wait()
                if q >= 1:
                    sd[q - 1].wait()
                if q < ch_rows - 1:
                    gd[q + 1] = pltpu.async_copy(
                        src_tbl.at[srcv.at[sl, q + 1]], rows.at[1 - b],
                        gsem.at[1 - b])
                sd[q] = pltpu.async_copy(
                    rows.at[b], table.at[dstv.at[sl, q]], ssem.at[b],
                    add=True)
                if q == 1:
                    @pl.when(k + 1 < nj)
                    def _():
                        load_idx(k + 1, 1 - sl)
            sd[ch_rows - 1].wait()
            return carry

        lax.fori_loop(0, nj, chunk_body, 0)
        plsc.subcore_barrier()

        pltpu.sync_copy(
            table.at[pl.ds(s * TILE_ROWS, TILE_ROWS)],
            out.at[pl.ds(c * N_NODES + s * TILE_ROWS, TILE_ROWS)])

        @pl.when(s == 0)
        def _():
            pltpu.sync_copy(
                table.at[pl.ds(NS * TILE_ROWS, TAIL_ROWS)],
                out.at[pl.ds(c * N_NODES + NS * TILE_ROWS, TAIL_ROWS)])

    return seg_sum


_seg_sum_l0 = _make_seg_sum(nj=5, ch_rows=16, src_stride=1280,
                            dst_stride=1280)
_seg_sum_l1 = _make_seg_sum(nj=5, ch_rows=32, src_stride=2560,
                            dst_stride=0)

_ROW_BLK = 1000


def _mlp0_body(s_ref, x_ref, a_ref, w1_ref, b1_ref, w2_ref, b2_ref, o_ref):
    rst = x_ref[...] * s_ref[0, 0] + a_ref[0] + a_ref[1]
    z = jnp.maximum(
        jnp.dot(rst, w1_ref[...], preferred_element_type=jnp.float32)
        + b1_ref[...], 0.0)
    o_ref[...] = (
        jnp.dot(z, w2_ref[...], preferred_element_type=jnp.float32)
        + b2_ref[...])


def _mlp0(scale, x, agg, W1, b1, W2, b2):
    grid = (N_NODES // _ROW_BLK,)
    return pl.pallas_call(
        _mlp0_body,
        grid=grid,
        in_specs=[
            pl.BlockSpec(memory_space=pltpu.SMEM),
            pl.BlockSpec((_ROW_BLK, 128), lambda i: (i, 0)),
            pl.BlockSpec((2, _ROW_BLK, 128), lambda i: (0, i, 0)),
            pl.BlockSpec((128, 256), lambda i: (0, 0)),
            pl.BlockSpec((1, 256), lambda i: (0, 0)),
            pl.BlockSpec((256, 256), lambda i: (0, 0)),
            pl.BlockSpec((1, 256), lambda i: (0, 0)),
        ],
        out_specs=pl.BlockSpec((_ROW_BLK, 256), lambda i: (i, 0)),
        out_shape=jax.ShapeDtypeStruct((N_NODES, 256), jnp.float32),
    )(scale, x, agg, W1, b1, W2, b2)


def _mlp1_body(s_ref, h_ref, a_ref, w1_ref, b1_ref, w2_ref, b2_ref, o_ref,
               acc_ref):
    i = pl.program_id(0)
    sc = s_ref[0, 0]
    rst = jnp.concatenate(
        [h_ref[:, :128] * sc + a_ref[0], h_ref[:, 128:] * sc + a_ref[1]],
        axis=1)
    z = jnp.maximum(
        jnp.dot(rst, w1_ref[...], preferred_element_type=jnp.float32)
        + b1_ref[...], 0.0)
    part = jnp.sum(z, axis=0, keepdims=True)

    @pl.when(i == 0)
    def _():
        acc_ref[...] = part

    @pl.when(i > 0)
    def _():
        acc_ref[...] += part

    @pl.when(i == pl.num_programs(0) - 1)
    def _():
        o_ref[...] = (
            jnp.dot(acc_ref[...], w2_ref[...],
                    preferred_element_type=jnp.float32)
            + b2_ref[...] * float(N_NODES))


def _mlp1(scale, h, agg, W1, b1, W2, b2):
    grid = (N_NODES // _ROW_BLK,)
    return pl.pallas_call(
        _mlp1_body,
        grid=grid,
        in_specs=[
            pl.BlockSpec(memory_space=pltpu.SMEM),
            pl.BlockSpec((_ROW_BLK, 256), lambda i: (i, 0)),
            pl.BlockSpec((2, _ROW_BLK, 128), lambda i: (0, i, 0)),
            pl.BlockSpec((256, 256), lambda i: (0, 0)),
            pl.BlockSpec((1, 256), lambda i: (0, 0)),
            pl.BlockSpec((256, 256), lambda i: (0, 0)),
            pl.BlockSpec((1, 256), lambda i: (0, 0)),
        ],
        out_specs=pl.BlockSpec((1, 256), lambda i: (0, 0)),
        out_shape=jax.ShapeDtypeStruct((1, 256), jnp.float32),
        scratch_shapes=[pltpu.VMEM((1, 256), jnp.float32)],
    )(scale, h, agg, W1, b1, W2, b2)


def kernel(x, edge_index, eps0, W1_0, b1_0, W2_0, b2_0,
           eps1, W1_1, b1_1, W2_1, b2_1):
    ei = edge_index.astype(jnp.int32)
    ar = jnp.arange(PAD, dtype=jnp.int32)
    src = jnp.concatenate([ei[0], ar % N_NODES])
    dst = jnp.concatenate([ei[1], N_NODES + (ar % 128)])
    src2d = src.reshape(-1, 128)
    dst2d = dst.reshape(-1, 128)

    agg0 = _seg_sum_l0(x, src2d, dst2d).reshape(NC, N_NODES, 128)
    scale0 = (1.0 + eps0).astype(jnp.float32).reshape(1, 1)
    h = _mlp0(scale0, x, agg0, W1_0, b1_0.reshape(1, 256),
              W2_0, b2_0.reshape(1, 256))

    hv = h.reshape(2 * N_NODES, 128)
    src2 = jnp.concatenate([src * 2, src * 2 + 1]).reshape(-1, 128)
    agg1 = _seg_sum_l1(hv, src2, dst2d).reshape(NC, N_NODES, 128)
    scale1 = (1.0 + eps1).astype(jnp.float32).reshape(1, 1)
    logits = _mlp1(scale1, h, agg1, W1_1, b1_1.reshape(1, 256),
                   W2_1, b2_1.reshape(1, 256))
    return logits
